# baseline (speedup 1.0000x reference)
import jax
import jax.numpy as jnp
from jax.experimental import pallas as pl
from jax.experimental.pallas import tpu as pltpu


def _fused_body(x_ref, wc_ref, bc_ref, w2_ref, b2_ref, o_ref, *, h, e):
    xb = x_ref[...]
    zc = jnp.dot(xb, wc_ref[...], preferred_element_type=jnp.float32) + bc_ref[...]
    z = zc[:, :h]
    lab = zc[:, h:h + e]
    y = jnp.argmax(lab, axis=-1)[:, None]
    out = jnp.dot(z, w2_ref[...], preferred_element_type=jnp.float32) + b2_ref[...]
    tn, es = o_ref.shape
    s = es // e
    col_expert = jax.lax.broadcasted_iota(jnp.int32, (tn, es), 1) // s
    o_ref[...] = jnp.where(col_expert == y, out, 0.0)


def kernel(x, W_lab, b_lab, W_enc, b_enc, W_clf, b_clf):
    N, D = x.shape
    E, H, S = W_clf.shape
    ES = E * S
    W2 = W_clf.transpose(1, 0, 2).reshape(H, ES)
    b2 = b_clf.reshape(1, ES)
    Wc = jnp.concatenate([W_enc, W_lab], axis=1)
    bc = jnp.concatenate([b_enc, b_lab]).reshape(1, H + E)

    TN = 2048
    grid = (N // TN,)

    import functools
    body = functools.partial(_fused_body, h=H, e=E)

    out = pl.pallas_call(
        body,
        grid=grid,
        in_specs=[
            pl.BlockSpec((TN, D), lambda i: (i, 0)),
            pl.BlockSpec((D, H + E), lambda i: (0, 0)),
            pl.BlockSpec((1, H + E), lambda i: (0, 0)),
            pl.BlockSpec((H, ES), lambda i: (0, 0)),
            pl.BlockSpec((1, ES), lambda i: (0, 0)),
        ],
        out_specs=pl.BlockSpec((TN, ES), lambda i: (i, 0)),
        out_shape=jax.ShapeDtypeStruct((N, ES), x.dtype),
        compiler_params=pltpu.CompilerParams(
            dimension_semantics=("parallel",),
        ),
    )(x, Wc, bc, W2, b2)
    return out


# speedup vs baseline: 1.0555x; 1.0555x over previous
import jax
import jax.numpy as jnp
from jax.experimental import pallas as pl
from jax.experimental.pallas import tpu as pltpu


def _fused_body(x_ref, wc_ref, bc_ref, w2_ref, b2_ref, o_ref, *, h, e):
    xb = x_ref[...]
    zc = jnp.dot(xb, wc_ref[...], preferred_element_type=jnp.float32) + bc_ref[...]
    z = zc[:, :h]
    lab = zc[:, h:h + e]
    y = jnp.argmax(lab, axis=-1)[:, None]
    out = jnp.dot(z, w2_ref[...], preferred_element_type=jnp.float32) + b2_ref[...]
    tn, es = o_ref.shape
    s = es // e
    col_expert = jax.lax.broadcasted_iota(jnp.int32, (tn, es), 1) // s
    o_ref[...] = jnp.where(col_expert == y, out, 0.0)


def kernel(x, W_lab, b_lab, W_enc, b_enc, W_clf, b_clf):
    N, D = x.shape
    E, H, S = W_clf.shape
    ES = E * S
    W2 = W_clf.transpose(1, 0, 2).reshape(H, ES)
    b2 = b_clf.reshape(1, ES)
    Wc = jnp.concatenate([W_enc, W_lab], axis=1)
    bc = jnp.concatenate([b_enc, b_lab]).reshape(1, H + E)

    TN = 4096
    grid = (N // TN,)

    import functools
    body = functools.partial(_fused_body, h=H, e=E)

    out = pl.pallas_call(
        body,
        grid=grid,
        in_specs=[
            pl.BlockSpec((TN, D), lambda i: (i, 0)),
            pl.BlockSpec((D, H + E), lambda i: (0, 0)),
            pl.BlockSpec((1, H + E), lambda i: (0, 0)),
            pl.BlockSpec((H, ES), lambda i: (0, 0)),
            pl.BlockSpec((1, ES), lambda i: (0, 0)),
        ],
        out_specs=pl.BlockSpec((TN, ES), lambda i: (i, 0)),
        out_shape=jax.ShapeDtypeStruct((N, ES), x.dtype),
        compiler_params=pltpu.CompilerParams(
            dimension_semantics=("parallel",),
        ),
    )(x, Wc, bc, W2, b2)
    return out
